# baseline (speedup 1.0000x reference)
import functools

import jax
import jax.numpy as jnp
from jax import lax
from jax.experimental import pallas as pl
from jax.experimental.pallas import tpu as pltpu

S_SIGMA = 8
N_BINS = 16
EPS = 1e-8
STRIPE = 128


def _shift(a, off, axis):
    n = a.shape[axis]
    if off == 0:
        return a
    zshape = list(a.shape)
    zshape[axis] = abs(off)
    z = jnp.zeros(zshape, a.dtype)
    if off > 0:
        body = lax.slice_in_dim(a, off, n, axis=axis)
        return jnp.concatenate([body, z], axis=axis)
    else:
        body = lax.slice_in_dim(a, 0, n + off, axis=axis)
        return jnp.concatenate([z, body], axis=axis)


def _blur_axis(a, taps, axis):
    acc = taps[2] * a
    for i in (0, 1, 3, 4):
        acc = acc + taps[i] * _shift(a, i - 2, axis)
    return acc


def _rep8_rows(a):
    R, W = a.shape
    return jnp.broadcast_to(a.reshape(R, 1, W), (R, 8, W)).reshape(8 * R, W)


def _bilateral_kernel(img_ref, fs_ref, fr_ref, out_ref, val_ref, wt_ref):
    H, W = img_ref.shape[1], img_ref.shape[2]
    GH = (H - 1) // S_SIGMA + 2
    GW = (W - 1) // S_SIGMA + 2
    GZ = N_BINS + 1
    f32 = jnp.float32

    img = img_ref[0]

    def fiota(shape, dim):
        return lax.broadcasted_iota(jnp.int32, shape, dim).astype(f32)

    ix = fiota((W, GW), 0)
    gj = fiota((W, GW), 1)
    SxT = (jnp.round(ix / S_SIGMA) == gj).astype(f32)
    SxT_main = SxT[:, 0:GW - 1]
    jx = fiota((GW, W), 0)
    px = fiota((GW, W), 1)
    WxT = jnp.maximum(0.0, 1.0 - jnp.abs(px / S_SIGMA - jx))
    NT = H // 128
    LR = 128 // S_SIGMA + 1
    lr = fiota((LR, 128), 1)
    ll = fiota((LR, 128), 0)
    SyP = (jnp.round(lr / S_SIGMA) == ll).astype(f32)
    gg = fiota((GH, NT * LR), 0)
    cc = fiota((GH, NT * LR), 1)
    Bmat = (gg == cc - jnp.floor(cc / LR)).astype(f32)

    gzf = jnp.clip(jnp.round(img * (N_BINS - 1)), 0.0, N_BINS - 1.0)
    for z in range(N_BINS):
        mask = (gzf == float(z)).astype(f32)
        mv = img * mask
        pv = jnp.concatenate(
            [jnp.dot(SyP, mv[128 * t:128 * (t + 1), :],
                     preferred_element_type=f32) for t in range(NT)], axis=0)
        pw = jnp.concatenate(
            [jnp.dot(SyP, mask[128 * t:128 * (t + 1), :],
                     preferred_element_type=f32) for t in range(NT)], axis=0)
        for p, ref in ((pv, val_ref), (pw, wt_ref)):
            a = jnp.dot(p, SxT, preferred_element_type=f32)
            ref[z, :, :] = jnp.dot(Bmat, a, preferred_element_type=f32)
    val_ref[N_BINS, :, :] = jnp.zeros((GH, GW), f32)
    wt_ref[N_BINS, :, :] = jnp.zeros((GH, GW), f32)

    fs = [fs_ref[i] for i in range(5)]
    fr = [fr_ref[i] for i in range(5)]
    for ref in (val_ref, wt_ref):
        a = ref[...]
        a = _blur_axis(a, fs, 1)
        a = _blur_axis(a, fs, 2)
        a = _blur_axis(a, fr, 0)
        ref[...] = a

    QROWS = STRIPE // S_SIGMA + 1
    rr = fiota((STRIPE, QROWS), 0)
    qq = fiota((STRIPE, QROWS), 1)
    Wy_s = jnp.maximum(0.0, 1.0 - jnp.abs(rr / S_SIGMA - qq))
    WxT_main = WxT[0:GW - 1, :]
    WxT_last = WxT[GW - 1:GW, :]
    for s in range(H // STRIPE):
        img_s = img_ref[0, s * STRIPE:(s + 1) * STRIPE, :]
        fz = jnp.clip(img_s * (N_BINS - 1), 0.0, N_BINS - 1.0)
        q0 = s * (STRIPE // S_SIGMA)
        accv = jnp.zeros((STRIPE, W), f32)
        accw = jnp.zeros((STRIPE, W), f32)
        for z in range(GZ):
            sv = val_ref[z, q0:q0 + QROWS, :]
            sw = wt_ref[z, q0:q0 + QROWS, :]
            vy = jnp.dot(Wy_s, sv, preferred_element_type=f32)
            wy = jnp.dot(Wy_s, sw, preferred_element_type=f32)
            vx = (jnp.dot(vy[:, 0:GW - 1], WxT_main, preferred_element_type=f32)
                  + vy[:, GW - 1:GW] * WxT_last)
            wx = (jnp.dot(wy[:, 0:GW - 1], WxT_main, preferred_element_type=f32)
                  + wy[:, GW - 1:GW] * WxT_last)
            tent = jnp.maximum(0.0, 1.0 - jnp.abs(fz - float(z)))
            accv = accv + tent * vx
            accw = accw + tent * wx
        out_ref[0, s * STRIPE:(s + 1) * STRIPE, :] = accv / (accw + EPS)


@functools.partial(jax.jit, static_argnames=("interpret",))
def _run(imgs, filter_s, filter_r, interpret=False):
    N, H, W = imgs.shape
    GH = (H - 1) // S_SIGMA + 2
    GW = (W - 1) // S_SIGMA + 2
    GZ = N_BINS + 1
    return pl.pallas_call(
        _bilateral_kernel,
        grid=(N,),
        in_specs=[
            pl.BlockSpec((1, H, W), lambda i: (i, 0, 0)),
            pl.BlockSpec(memory_space=pltpu.SMEM),
            pl.BlockSpec(memory_space=pltpu.SMEM),
        ],
        out_specs=pl.BlockSpec((1, H, W), lambda i: (i, 0, 0)),
        out_shape=jax.ShapeDtypeStruct((N, H, W), jnp.float32),
        scratch_shapes=[
            pltpu.VMEM((GZ, GH, GW), jnp.float32),
            pltpu.VMEM((GZ, GH, GW), jnp.float32),
        ],
        compiler_params=pltpu.CompilerParams(
            dimension_semantics=("parallel",)),
        interpret=interpret,
    )(imgs, filter_s, filter_r)


def kernel(blurred_batch, kernel_batch, filter_s, filter_r, num_irls_iter, num_cg_iter):
    B, C, H, W = blurred_batch.shape
    imgs = blurred_batch.reshape(B * C, H, W)
    out = _run(imgs, filter_s, filter_r)
    return out.reshape(B, C, H, W)


# speedup vs baseline: 1.1135x; 1.1135x over previous
import functools

import jax
import jax.numpy as jnp
from jax import lax
from jax.experimental import pallas as pl
from jax.experimental.pallas import tpu as pltpu

S_SIGMA = 8
N_BINS = 16
EPS = 1e-8
STRIPE = 128


def _shift(a, off, axis):
    n = a.shape[axis]
    if off == 0:
        return a
    zshape = list(a.shape)
    zshape[axis] = abs(off)
    z = jnp.zeros(zshape, a.dtype)
    if off > 0:
        body = lax.slice_in_dim(a, off, n, axis=axis)
        return jnp.concatenate([body, z], axis=axis)
    else:
        body = lax.slice_in_dim(a, 0, n + off, axis=axis)
        return jnp.concatenate([z, body], axis=axis)


def _blur_axis(a, taps, axis):
    acc = taps[2] * a
    for i in (0, 1, 3, 4):
        acc = acc + taps[i] * _shift(a, i - 2, axis)
    return acc


def _rep8_rows(a):
    R, W = a.shape
    return jnp.broadcast_to(a.reshape(R, 1, W), (R, 8, W)).reshape(8 * R, W)


def _bilateral_kernel(img_ref, fs_ref, fr_ref, out_ref, val_ref, wt_ref):
    H, W = img_ref.shape[1], img_ref.shape[2]
    GH = (H - 1) // S_SIGMA + 2
    GW = (W - 1) // S_SIGMA + 2
    GZ = N_BINS + 1
    f32 = jnp.float32

    img = img_ref[0]

    def fiota(shape, dim):
        return lax.broadcasted_iota(jnp.int32, shape, dim).astype(f32)

    ix = fiota((W, GW), 0)
    gj = fiota((W, GW), 1)
    SxT = (jnp.round(ix / S_SIGMA) == gj).astype(f32)
    SxT_main = SxT[:, 0:GW - 1]
    jx = fiota((GW, W), 0)
    px = fiota((GW, W), 1)
    WxT = jnp.maximum(0.0, 1.0 - jnp.abs(px / S_SIGMA - jx))
    NT = H // 128
    LR = 128 // S_SIGMA + 1
    lr = fiota((LR, 128), 1)
    ll = fiota((LR, 128), 0)
    SyP = (jnp.round(lr / S_SIGMA) == ll).astype(f32)
    gg = fiota((GH, NT * LR), 0)
    cc = fiota((GH, NT * LR), 1)
    Bmat = (gg == cc - jnp.floor(cc / LR)).astype(f32)

    gzf = jnp.clip(jnp.round(img * (N_BINS - 1)), 0.0, N_BINS - 1.0)
    for z in range(N_BINS):
        mask = (gzf == float(z)).astype(f32)
        mv = img * mask
        pv = jnp.concatenate(
            [jnp.dot(SyP, mv[128 * t:128 * (t + 1), :],
                     preferred_element_type=f32) for t in range(NT)], axis=0)
        pw = jnp.concatenate(
            [jnp.dot(SyP, mask[128 * t:128 * (t + 1), :],
                     preferred_element_type=f32) for t in range(NT)], axis=0)
        for p, ref in ((pv, val_ref), (pw, wt_ref)):
            a = jnp.dot(p, SxT, preferred_element_type=f32)
            ref[z, :, :] = jnp.dot(Bmat, a, preferred_element_type=f32)
    val_ref[N_BINS, :, :] = jnp.zeros((GH, GW), f32)
    wt_ref[N_BINS, :, :] = jnp.zeros((GH, GW), f32)

    fs = [fs_ref[i] for i in range(5)]
    fr = [fr_ref[i] for i in range(5)]
    for ref in (val_ref, wt_ref):
        a = ref[...]
        a = _blur_axis(a, fs, 1)
        a = _blur_axis(a, fs, 2)
        a = _blur_axis(a, fr, 0)
        ref[...] = a

    QROWS = STRIPE // S_SIGMA + 1
    rr = fiota((STRIPE, QROWS), 0)
    qq = fiota((STRIPE, QROWS), 1)
    Wy_s = jnp.maximum(0.0, 1.0 - jnp.abs(rr / S_SIGMA - qq))
    WxT_main = WxT[0:GW - 1, :]
    WxT_last = WxT[GW - 1:GW, :]
    for s in range(H // STRIPE):
        img_s = img_ref[0, s * STRIPE:(s + 1) * STRIPE, :]
        fz = jnp.clip(img_s * (N_BINS - 1), 0.0, N_BINS - 1.0)
        q0 = s * (STRIPE // S_SIGMA)
        vy = jnp.concatenate(
            [jnp.dot(Wy_s, val_ref[z, q0:q0 + QROWS, :],
                     preferred_element_type=f32) for z in range(GZ)], axis=0)
        wy = jnp.concatenate(
            [jnp.dot(Wy_s, wt_ref[z, q0:q0 + QROWS, :],
                     preferred_element_type=f32) for z in range(GZ)], axis=0)
        vx = (jnp.dot(vy[:, 0:GW - 1], WxT_main, preferred_element_type=f32)
              + vy[:, GW - 1:GW] * WxT_last)
        wx = (jnp.dot(wy[:, 0:GW - 1], WxT_main, preferred_element_type=f32)
              + wy[:, GW - 1:GW] * WxT_last)
        accv = jnp.zeros((STRIPE, W), f32)
        accw = jnp.zeros((STRIPE, W), f32)
        for z in range(GZ):
            tent = jnp.maximum(0.0, 1.0 - jnp.abs(fz - float(z)))
            accv = accv + tent * vx[z * STRIPE:(z + 1) * STRIPE, :]
            accw = accw + tent * wx[z * STRIPE:(z + 1) * STRIPE, :]
        out_ref[0, s * STRIPE:(s + 1) * STRIPE, :] = accv / (accw + EPS)


@functools.partial(jax.jit, static_argnames=("interpret",))
def _run(imgs, filter_s, filter_r, interpret=False):
    N, H, W = imgs.shape
    GH = (H - 1) // S_SIGMA + 2
    GW = (W - 1) // S_SIGMA + 2
    GZ = N_BINS + 1
    return pl.pallas_call(
        _bilateral_kernel,
        grid=(N,),
        in_specs=[
            pl.BlockSpec((1, H, W), lambda i: (i, 0, 0)),
            pl.BlockSpec(memory_space=pltpu.SMEM),
            pl.BlockSpec(memory_space=pltpu.SMEM),
        ],
        out_specs=pl.BlockSpec((1, H, W), lambda i: (i, 0, 0)),
        out_shape=jax.ShapeDtypeStruct((N, H, W), jnp.float32),
        scratch_shapes=[
            pltpu.VMEM((GZ, GH, GW), jnp.float32),
            pltpu.VMEM((GZ, GH, GW), jnp.float32),
        ],
        compiler_params=pltpu.CompilerParams(
            dimension_semantics=("parallel",)),
        interpret=interpret,
    )(imgs, filter_s, filter_r)


def kernel(blurred_batch, kernel_batch, filter_s, filter_r, num_irls_iter, num_cg_iter):
    B, C, H, W = blurred_batch.shape
    imgs = blurred_batch.reshape(B * C, H, W)
    out = _run(imgs, filter_s, filter_r)
    return out.reshape(B, C, H, W)
